# baseline (speedup 1.0000x reference)
import functools

import jax
import jax.numpy as jnp
from jax import lax
from jax.experimental import pallas as pl
from jax.experimental.pallas import tpu as pltpu
from jax.experimental.pallas import tpu_sc as plsc

S = 64
G = 128
P = G * G
F = 6
B = 16384
GXBLK = 32
SH = S // 2
GXBLK2 = 32
BBLK = 2048
NW = 32
BPW = B // NW
L = 16


def _vsum_body(sf_ref, w_ref, v_ref):
    v = w_ref[0, 0] * sf_ref[0]
    for f in range(1, F):
        v = v + w_ref[0, f] * sf_ref[f]
    v_ref[...] = v


def _softmax_body(va_ref, vb_ref, t_ref):
    v = jnp.concatenate([va_ref[...], vb_ref[...]], axis=0)
    m = jnp.max(v, axis=0)
    e = jnp.exp(v - m[None])
    z = jnp.sum(e, axis=0)
    num = jnp.sum(e * v, axis=0)
    t_ref[...] = num / z


def _phi_body(p_ref, w_ref, pr_ref, idx_ref):
    pr_l = w_ref[0, 0] * p_ref[0:1]
    pr_r = w_ref[0, 0] * p_ref[10:11]
    for f in range(1, F):
        pr_l = pr_l + w_ref[0, f] * p_ref[f:f + 1]
        pr_r = pr_r + w_ref[0, f] * p_ref[10 + f:11 + f]
    pr_ref[...] = jnp.concatenate([pr_l, pr_r], axis=0)

    def flat_idx(xrow, yrow):
        return p_ref[xrow:xrow + 1].astype(jnp.int32) * G + \
               p_ref[yrow:yrow + 1].astype(jnp.int32)

    idx_ref[...] = jnp.concatenate(
        [flat_idx(6, 7),
         flat_idx(16, 17),
         flat_idx(8, 9),
         flat_idx(18, 19)],
        axis=0)


def _sc_combine_body(t_hbm, idx_hbm, pr_hbm, out_hbm,
                     t_v, idx_v, pr_v, out_v, sem):
    wid = lax.axis_index("s") * 2 + lax.axis_index("c")
    base = wid * BPW
    c1 = pltpu.async_copy(t_hbm, t_v, sem)
    c2 = pltpu.async_copy(idx_hbm.at[:, pl.ds(base, BPW)], idx_v, sem)
    c3 = pltpu.async_copy(pr_hbm.at[:, pl.ds(base, BPW)], pr_v, sem)
    c1.wait()
    c2.wait()
    c3.wait()

    def body(c, carry):
        s = c * L
        v_ssl = plsc.load_gather(t_v, [idx_v[0, pl.ds(s, L)]])
        v_ssr = plsc.load_gather(t_v, [idx_v[1, pl.ds(s, L)]])
        v_esl = plsc.load_gather(t_v, [idx_v[2, pl.ds(s, L)]])
        v_esr = plsc.load_gather(t_v, [idx_v[3, pl.ds(s, L)]])
        d_l = pr_v[0, pl.ds(s, L)] + v_esl - v_ssl
        d_r = pr_v[1, pl.ds(s, L)] + v_esr - v_ssr
        z = d_l - d_r
        out_v[0, pl.ds(s, L)] = 1.0 / (1.0 + jnp.exp(-z))
        out_v[1, pl.ds(s, L)] = 1.0 / (1.0 + jnp.exp(z))
        return carry

    lax.fori_loop(0, BPW // L, body, 0)
    pltpu.sync_copy(out_v, out_hbm.at[:, pl.ds(base, BPW)])


@functools.cache
def _make_sc_combine():
    return functools.partial(
        pl.kernel,
        mesh=plsc.VectorSubcoreMesh(core_axis_name="c", subcore_axis_name="s"),
        out_type=jax.ShapeDtypeStruct((2, B), jnp.float32),
        compiler_params=pltpu.CompilerParams(needs_layout_passes=False),
        scratch_types=[
            pltpu.VMEM((P,), jnp.float32),
            pltpu.VMEM((4, BPW), jnp.int32),
            pltpu.VMEM((2, BPW), jnp.float32),
            pltpu.VMEM((2, BPW), jnp.float32),
            pltpu.SemaphoreType.DMA,
        ],
    )(_sc_combine_body)


def kernel(phi, succ_feats, W):
    sfT_a = jnp.transpose(succ_feats[:SH], (3, 0, 1, 2))
    sfT_b = jnp.transpose(succ_feats[SH:], (3, 0, 1, 2))
    phiT = jnp.transpose(phi.reshape(B, 2 * 10), (1, 0))

    def vsum(sfT_half):
        return pl.pallas_call(
            _vsum_body,
            grid=(G // GXBLK,),
            in_specs=[
                pl.BlockSpec((F, SH, GXBLK, G), lambda j: (0, 0, j, 0)),
                pl.BlockSpec(memory_space=pltpu.SMEM),
            ],
            out_specs=pl.BlockSpec((SH, GXBLK, G), lambda j: (0, j, 0)),
            out_shape=jax.ShapeDtypeStruct((SH, G, G), jnp.float32),
        )(sfT_half, W)

    va = vsum(sfT_a)
    vb = vsum(sfT_b)

    t_tab = pl.pallas_call(
        _softmax_body,
        grid=(G // GXBLK2,),
        in_specs=[
            pl.BlockSpec((SH, GXBLK2, G), lambda j: (0, j, 0)),
            pl.BlockSpec((SH, GXBLK2, G), lambda j: (0, j, 0)),
        ],
        out_specs=pl.BlockSpec((GXBLK2, G), lambda j: (j, 0)),
        out_shape=jax.ShapeDtypeStruct((G, G), jnp.float32),
    )(va, vb)

    pr, idx = pl.pallas_call(
        _phi_body,
        grid=(B // BBLK,),
        in_specs=[
            pl.BlockSpec((20, BBLK), lambda j: (0, j)),
            pl.BlockSpec(memory_space=pltpu.SMEM),
        ],
        out_specs=[
            pl.BlockSpec((2, BBLK), lambda j: (0, j)),
            pl.BlockSpec((4, BBLK), lambda j: (0, j)),
        ],
        out_shape=[
            jax.ShapeDtypeStruct((2, B), jnp.float32),
            jax.ShapeDtypeStruct((4, B), jnp.int32),
        ],
    )(phiT, W)

    out = _make_sc_combine()(t_tab.reshape(P), idx, pr)
    return jnp.transpose(out, (1, 0))[:, :, None]


# speedup vs baseline: 1.3231x; 1.3231x over previous
import functools

import jax
import jax.numpy as jnp
from jax import lax
from jax.experimental import pallas as pl
from jax.experimental.pallas import tpu as pltpu
from jax.experimental.pallas import tpu_sc as plsc

S = 64
G = 128
P = G * G
F = 6
B = 16384
GXBLK = 32
BBLK = 2048
NW = 32
BPW = B // NW
L = 16


def _table_body(sf_ref, w_ref, t_ref):
    v = w_ref[0, 0] * sf_ref[0]
    for f in range(1, F):
        v = v + w_ref[0, f] * sf_ref[f]
    m = jnp.max(v, axis=0)
    e = jnp.exp(v - m[None])
    z = jnp.sum(e, axis=0)
    num = jnp.sum(e * v, axis=0)
    t_ref[...] = num / z


def _phi_body(p_ref, w_ref, pr_ref, idx_ref):
    pr_l = w_ref[0, 0] * p_ref[0:1]
    pr_r = w_ref[0, 0] * p_ref[10:11]
    for f in range(1, F):
        pr_l = pr_l + w_ref[0, f] * p_ref[f:f + 1]
        pr_r = pr_r + w_ref[0, f] * p_ref[10 + f:11 + f]
    pr_ref[...] = jnp.concatenate([pr_l, pr_r], axis=0)

    def flat_idx(xrow, yrow):
        return p_ref[xrow:xrow + 1].astype(jnp.int32) * G + \
               p_ref[yrow:yrow + 1].astype(jnp.int32)

    idx_ref[...] = jnp.concatenate(
        [flat_idx(6, 7),
         flat_idx(16, 17),
         flat_idx(8, 9),
         flat_idx(18, 19)],
        axis=0)


def _sc_combine_body(t_hbm, idx_hbm, pr_hbm, out_hbm,
                     t_v, idx_v, pr_v, out_v, sem):
    wid = lax.axis_index("s") * 2 + lax.axis_index("c")
    base = wid * BPW
    c1 = pltpu.async_copy(t_hbm, t_v, sem)
    c2 = pltpu.async_copy(idx_hbm.at[:, pl.ds(base, BPW)], idx_v, sem)
    c3 = pltpu.async_copy(pr_hbm.at[:, pl.ds(base, BPW)], pr_v, sem)
    c1.wait()
    c2.wait()
    c3.wait()

    def body(c, carry):
        s = c * L
        v_ssl = plsc.load_gather(t_v, [idx_v[0, pl.ds(s, L)]])
        v_ssr = plsc.load_gather(t_v, [idx_v[1, pl.ds(s, L)]])
        v_esl = plsc.load_gather(t_v, [idx_v[2, pl.ds(s, L)]])
        v_esr = plsc.load_gather(t_v, [idx_v[3, pl.ds(s, L)]])
        d_l = pr_v[0, pl.ds(s, L)] + v_esl - v_ssl
        d_r = pr_v[1, pl.ds(s, L)] + v_esr - v_ssr
        z = d_l - d_r
        out_v[0, pl.ds(s, L)] = 1.0 / (1.0 + jnp.exp(-z))
        out_v[1, pl.ds(s, L)] = 1.0 / (1.0 + jnp.exp(z))
        return carry

    lax.fori_loop(0, BPW // L, body, 0)
    pltpu.sync_copy(out_v, out_hbm.at[:, pl.ds(base, BPW)])


@functools.cache
def _make_sc_combine():
    return functools.partial(
        pl.kernel,
        mesh=plsc.VectorSubcoreMesh(core_axis_name="c", subcore_axis_name="s"),
        out_type=jax.ShapeDtypeStruct((2, B), jnp.float32),
        compiler_params=pltpu.CompilerParams(needs_layout_passes=False),
        scratch_types=[
            pltpu.VMEM((P,), jnp.float32),
            pltpu.VMEM((4, BPW), jnp.int32),
            pltpu.VMEM((2, BPW), jnp.float32),
            pltpu.VMEM((2, BPW), jnp.float32),
            pltpu.SemaphoreType.DMA,
        ],
    )(_sc_combine_body)


def kernel(phi, succ_feats, W):
    sfT = jnp.transpose(succ_feats, (3, 0, 1, 2))
    phiT = jnp.transpose(phi.reshape(B, 2 * 10), (1, 0))

    t_tab = pl.pallas_call(
        _table_body,
        grid=(G // GXBLK,),
        in_specs=[
            pl.BlockSpec((F, S, GXBLK, G), lambda j: (0, 0, j, 0)),
            pl.BlockSpec(memory_space=pltpu.SMEM),
        ],
        out_specs=pl.BlockSpec((GXBLK, G), lambda j: (j, 0)),
        out_shape=jax.ShapeDtypeStruct((G, G), jnp.float32),
    )(sfT, W)

    pr, idx = pl.pallas_call(
        _phi_body,
        grid=(B // BBLK,),
        in_specs=[
            pl.BlockSpec((20, BBLK), lambda j: (0, j)),
            pl.BlockSpec(memory_space=pltpu.SMEM),
        ],
        out_specs=[
            pl.BlockSpec((2, BBLK), lambda j: (0, j)),
            pl.BlockSpec((4, BBLK), lambda j: (0, j)),
        ],
        out_shape=[
            jax.ShapeDtypeStruct((2, B), jnp.float32),
            jax.ShapeDtypeStruct((4, B), jnp.int32),
        ],
    )(phiT, W)

    out = _make_sc_combine()(t_tab.reshape(P), idx, pr)
    return jnp.transpose(out, (1, 0))[:, :, None]


# speedup vs baseline: 1.4195x; 1.0728x over previous
import functools

import jax
import jax.numpy as jnp
from jax import lax
from jax.experimental import pallas as pl
from jax.experimental.pallas import tpu as pltpu
from jax.experimental.pallas import tpu_sc as plsc

S = 64
G = 128
P = G * G
F = 6
B = 16384
GXBLK = 32
BBLK = 2048
NW = 32
BPW = B // NW
L = 16


def _cast_body(sf_ref, o_ref):
    o_ref[...] = sf_ref[...].astype(jnp.bfloat16)


def _table_body(sf_ref, w_ref, t_ref):
    v = w_ref[0, 0] * sf_ref[0].astype(jnp.float32)
    for f in range(1, F):
        v = v + w_ref[0, f] * sf_ref[f].astype(jnp.float32)
    m = jnp.max(v, axis=0)
    e = jnp.exp(v - m[None])
    z = jnp.sum(e, axis=0)
    num = jnp.sum(e * v, axis=0)
    t_ref[...] = num / z


def _phi_body(p_ref, w_ref, pr_ref, idx_ref):
    pr_l = w_ref[0, 0] * p_ref[0:1]
    pr_r = w_ref[0, 0] * p_ref[10:11]
    for f in range(1, F):
        pr_l = pr_l + w_ref[0, f] * p_ref[f:f + 1]
        pr_r = pr_r + w_ref[0, f] * p_ref[10 + f:11 + f]
    pr_ref[...] = jnp.concatenate([pr_l, pr_r], axis=0)

    def flat_idx(xrow, yrow):
        return p_ref[xrow:xrow + 1].astype(jnp.int32) * G + \
               p_ref[yrow:yrow + 1].astype(jnp.int32)

    idx_ref[...] = jnp.concatenate(
        [flat_idx(6, 7),
         flat_idx(16, 17),
         flat_idx(8, 9),
         flat_idx(18, 19)],
        axis=0)


def _sc_combine_body(t_hbm, idx_hbm, pr_hbm, out_hbm,
                     t_v, idx_v, pr_v, out_v, sem):
    wid = lax.axis_index("s") * 2 + lax.axis_index("c")
    base = wid * BPW
    c1 = pltpu.async_copy(t_hbm, t_v, sem)
    c2 = pltpu.async_copy(idx_hbm.at[:, pl.ds(base, BPW)], idx_v, sem)
    c3 = pltpu.async_copy(pr_hbm.at[:, pl.ds(base, BPW)], pr_v, sem)
    c1.wait()
    c2.wait()
    c3.wait()

    def body(c, carry):
        s = c * L
        v_ssl = plsc.load_gather(t_v, [idx_v[0, pl.ds(s, L)]])
        v_ssr = plsc.load_gather(t_v, [idx_v[1, pl.ds(s, L)]])
        v_esl = plsc.load_gather(t_v, [idx_v[2, pl.ds(s, L)]])
        v_esr = plsc.load_gather(t_v, [idx_v[3, pl.ds(s, L)]])
        d_l = pr_v[0, pl.ds(s, L)] + v_esl - v_ssl
        d_r = pr_v[1, pl.ds(s, L)] + v_esr - v_ssr
        z = d_l - d_r
        out_v[0, pl.ds(s, L)] = 1.0 / (1.0 + jnp.exp(-z))
        out_v[1, pl.ds(s, L)] = 1.0 / (1.0 + jnp.exp(z))
        return carry

    lax.fori_loop(0, BPW // L, body, 0)
    pltpu.sync_copy(out_v, out_hbm.at[:, pl.ds(base, BPW)])


@functools.cache
def _make_sc_combine():
    return functools.partial(
        pl.kernel,
        mesh=plsc.VectorSubcoreMesh(core_axis_name="c", subcore_axis_name="s"),
        out_type=jax.ShapeDtypeStruct((2, B), jnp.float32),
        compiler_params=pltpu.CompilerParams(needs_layout_passes=False),
        scratch_types=[
            pltpu.VMEM((P,), jnp.float32),
            pltpu.VMEM((4, BPW), jnp.int32),
            pltpu.VMEM((2, BPW), jnp.float32),
            pltpu.VMEM((2, BPW), jnp.float32),
            pltpu.SemaphoreType.DMA,
        ],
    )(_sc_combine_body)


def kernel(phi, succ_feats, W):
    sfT = jnp.transpose(succ_feats.astype(jnp.bfloat16), (3, 0, 1, 2))
    phiT = jnp.transpose(phi.reshape(B, 2 * 10), (1, 0))

    t_tab = pl.pallas_call(
        _table_body,
        grid=(G // GXBLK,),
        in_specs=[
            pl.BlockSpec((F, S, GXBLK, G), lambda j: (0, 0, j, 0)),
            pl.BlockSpec(memory_space=pltpu.SMEM),
        ],
        out_specs=pl.BlockSpec((GXBLK, G), lambda j: (j, 0)),
        out_shape=jax.ShapeDtypeStruct((G, G), jnp.float32),
    )(sfT, W)

    pr, idx = pl.pallas_call(
        _phi_body,
        grid=(B // BBLK,),
        in_specs=[
            pl.BlockSpec((20, BBLK), lambda j: (0, j)),
            pl.BlockSpec(memory_space=pltpu.SMEM),
        ],
        out_specs=[
            pl.BlockSpec((2, BBLK), lambda j: (0, j)),
            pl.BlockSpec((4, BBLK), lambda j: (0, j)),
        ],
        out_shape=[
            jax.ShapeDtypeStruct((2, B), jnp.float32),
            jax.ShapeDtypeStruct((4, B), jnp.int32),
        ],
    )(phiT, W)

    out = _make_sc_combine()(t_tab.reshape(P), idx, pr)
    return jnp.transpose(out, (1, 0))[:, :, None]


# speedup vs baseline: 1.7710x; 1.2476x over previous
import functools

import jax
import jax.numpy as jnp
from jax import lax
from jax.experimental import pallas as pl
from jax.experimental.pallas import tpu as pltpu
from jax.experimental.pallas import tpu_sc as plsc

S = 64
G = 128
P = G * G
F = 6
B = 16384
GXBLK = 32
BBLK = 2048
NW = 32
BPW = B // NW
L = 16


def _table_body(sf_ref, w_ref, t_ref):
    v = w_ref[0, 0] * sf_ref[:, 0]
    for f in range(1, F):
        v = v + w_ref[0, f] * sf_ref[:, f]
    m = jnp.max(v, axis=0)
    e = jnp.exp(v - m[None])
    z = jnp.sum(e, axis=0)
    num = jnp.sum(e * v, axis=0)
    t_ref[...] = num / z


def _phi_body(p_ref, w_ref, pr_ref, idx_ref):
    pr_l = w_ref[0, 0] * p_ref[0:1]
    pr_r = w_ref[0, 0] * p_ref[10:11]
    for f in range(1, F):
        pr_l = pr_l + w_ref[0, f] * p_ref[f:f + 1]
        pr_r = pr_r + w_ref[0, f] * p_ref[10 + f:11 + f]
    pr_ref[...] = jnp.concatenate([pr_l, pr_r], axis=0)

    def flat_idx(xrow, yrow):
        return p_ref[xrow:xrow + 1].astype(jnp.int32) * G + \
               p_ref[yrow:yrow + 1].astype(jnp.int32)

    idx_ref[...] = jnp.concatenate(
        [flat_idx(6, 7),
         flat_idx(16, 17),
         flat_idx(8, 9),
         flat_idx(18, 19)],
        axis=0)


def _sc_combine_body(t_hbm, idx_hbm, pr_hbm, out_hbm,
                     t_v, idx_v, pr_v, out_v, sem):
    wid = lax.axis_index("s") * 2 + lax.axis_index("c")
    base = wid * BPW
    c1 = pltpu.async_copy(t_hbm, t_v, sem)
    c2 = pltpu.async_copy(idx_hbm.at[:, pl.ds(base, BPW)], idx_v, sem)
    c3 = pltpu.async_copy(pr_hbm.at[:, pl.ds(base, BPW)], pr_v, sem)
    c1.wait()
    c2.wait()
    c3.wait()

    def body(c, carry):
        s = c * L
        v_ssl = plsc.load_gather(t_v, [idx_v[0, pl.ds(s, L)]])
        v_ssr = plsc.load_gather(t_v, [idx_v[1, pl.ds(s, L)]])
        v_esl = plsc.load_gather(t_v, [idx_v[2, pl.ds(s, L)]])
        v_esr = plsc.load_gather(t_v, [idx_v[3, pl.ds(s, L)]])
        d_l = pr_v[0, pl.ds(s, L)] + v_esl - v_ssl
        d_r = pr_v[1, pl.ds(s, L)] + v_esr - v_ssr
        z = d_l - d_r
        out_v[0, pl.ds(s, L)] = 1.0 / (1.0 + jnp.exp(-z))
        out_v[1, pl.ds(s, L)] = 1.0 / (1.0 + jnp.exp(z))
        return carry

    lax.fori_loop(0, BPW // L, body, 0)
    pltpu.sync_copy(out_v, out_hbm.at[:, pl.ds(base, BPW)])


@functools.cache
def _make_sc_combine():
    return functools.partial(
        pl.kernel,
        mesh=plsc.VectorSubcoreMesh(core_axis_name="c", subcore_axis_name="s"),
        out_type=jax.ShapeDtypeStruct((2, B), jnp.float32),
        compiler_params=pltpu.CompilerParams(needs_layout_passes=False),
        scratch_types=[
            pltpu.VMEM((P,), jnp.float32),
            pltpu.VMEM((4, BPW), jnp.int32),
            pltpu.VMEM((2, BPW), jnp.float32),
            pltpu.VMEM((2, BPW), jnp.float32),
            pltpu.SemaphoreType.DMA,
        ],
    )(_sc_combine_body)


def kernel(phi, succ_feats, W):
    sfT = jnp.transpose(succ_feats, (0, 3, 1, 2))
    phiT = jnp.transpose(phi.reshape(B, 2 * 10), (1, 0))

    t_tab = pl.pallas_call(
        _table_body,
        grid=(G // GXBLK,),
        in_specs=[
            pl.BlockSpec((S, F, GXBLK, G), lambda j: (0, 0, j, 0)),
            pl.BlockSpec(memory_space=pltpu.SMEM),
        ],
        out_specs=pl.BlockSpec((GXBLK, G), lambda j: (j, 0)),
        out_shape=jax.ShapeDtypeStruct((G, G), jnp.float32),
    )(sfT, W)

    pr, idx = pl.pallas_call(
        _phi_body,
        grid=(B // BBLK,),
        in_specs=[
            pl.BlockSpec((20, BBLK), lambda j: (0, j)),
            pl.BlockSpec(memory_space=pltpu.SMEM),
        ],
        out_specs=[
            pl.BlockSpec((2, BBLK), lambda j: (0, j)),
            pl.BlockSpec((4, BBLK), lambda j: (0, j)),
        ],
        out_shape=[
            jax.ShapeDtypeStruct((2, B), jnp.float32),
            jax.ShapeDtypeStruct((4, B), jnp.int32),
        ],
    )(phiT, W)

    out = _make_sc_combine()(t_tab.reshape(P), idx, pr)
    return jnp.transpose(out, (1, 0))[:, :, None]


# speedup vs baseline: 1.7807x; 1.0055x over previous
import functools

import jax
import jax.numpy as jnp
from jax import lax
from jax.experimental import pallas as pl
from jax.experimental.pallas import tpu as pltpu
from jax.experimental.pallas import tpu_sc as plsc

S = 64
G = 128
P = G * G
F = 6
B = 16384
GXBLK = 32
BBLK = 2048
NW = 32
BPW = B // NW
L = 16


def _table_body(sf_ref, w_ref, t_ref):
    v = w_ref[0, 0] * sf_ref[:, 0]
    for f in range(1, F):
        v = v + w_ref[0, f] * sf_ref[:, f]
    m = jnp.max(v, axis=0)
    e = jnp.exp(v - m[None])
    z = jnp.sum(e, axis=0)
    num = jnp.sum(e * v, axis=0)
    t_ref[...] = num / z


def _phi_body(p_ref, w_ref, pr_ref, idx_ref):
    pr_l = w_ref[0, 0] * p_ref[0:1]
    pr_r = w_ref[0, 0] * p_ref[1:2]
    for f in range(1, F):
        pr_l = pr_l + w_ref[0, f] * p_ref[2 * f:2 * f + 1]
        pr_r = pr_r + w_ref[0, f] * p_ref[2 * f + 1:2 * f + 2]
    pr_ref[...] = jnp.concatenate([pr_l, pr_r], axis=0)

    def flat_idx(xrow, yrow):
        return p_ref[xrow:xrow + 1].astype(jnp.int32) * G + \
               p_ref[yrow:yrow + 1].astype(jnp.int32)

    idx_ref[...] = jnp.concatenate(
        [flat_idx(12, 14),
         flat_idx(13, 15),
         flat_idx(16, 18),
         flat_idx(17, 19)],
        axis=0)


def _sc_combine_body(t_hbm, idx_hbm, pr_hbm, out_hbm,
                     t_v, idx_v, pr_v, out_v, sem):
    wid = lax.axis_index("s") * 2 + lax.axis_index("c")
    base = wid * BPW
    c1 = pltpu.async_copy(t_hbm, t_v, sem)
    c2 = pltpu.async_copy(idx_hbm.at[:, pl.ds(base, BPW)], idx_v, sem)
    c3 = pltpu.async_copy(pr_hbm.at[:, pl.ds(base, BPW)], pr_v, sem)
    c1.wait()
    c2.wait()
    c3.wait()

    def body(c, carry):
        s = c * L
        v_ssl = plsc.load_gather(t_v, [idx_v[0, pl.ds(s, L)]])
        v_ssr = plsc.load_gather(t_v, [idx_v[1, pl.ds(s, L)]])
        v_esl = plsc.load_gather(t_v, [idx_v[2, pl.ds(s, L)]])
        v_esr = plsc.load_gather(t_v, [idx_v[3, pl.ds(s, L)]])
        d_l = pr_v[0, pl.ds(s, L)] + v_esl - v_ssl
        d_r = pr_v[1, pl.ds(s, L)] + v_esr - v_ssr
        z = d_l - d_r
        out_v[0, pl.ds(s, L)] = 1.0 / (1.0 + jnp.exp(-z))
        out_v[1, pl.ds(s, L)] = 1.0 / (1.0 + jnp.exp(z))
        return carry

    lax.fori_loop(0, BPW // L, body, 0)
    pltpu.sync_copy(out_v, out_hbm.at[:, pl.ds(base, BPW)])


@functools.cache
def _make_sc_combine():
    return functools.partial(
        pl.kernel,
        mesh=plsc.VectorSubcoreMesh(core_axis_name="c", subcore_axis_name="s"),
        out_type=jax.ShapeDtypeStruct((2, B), jnp.float32),
        compiler_params=pltpu.CompilerParams(needs_layout_passes=False),
        scratch_types=[
            pltpu.VMEM((P,), jnp.float32),
            pltpu.VMEM((4, BPW), jnp.int32),
            pltpu.VMEM((2, BPW), jnp.float32),
            pltpu.VMEM((2, BPW), jnp.float32),
            pltpu.SemaphoreType.DMA,
        ],
    )(_sc_combine_body)


def kernel(phi, succ_feats, W):
    sfT = jnp.transpose(succ_feats, (0, 3, 1, 2))
    phiT = jnp.transpose(phi, (2, 1, 0)).reshape(20, B)

    t_tab = pl.pallas_call(
        _table_body,
        grid=(G // GXBLK,),
        in_specs=[
            pl.BlockSpec((S, F, GXBLK, G), lambda j: (0, 0, j, 0)),
            pl.BlockSpec(memory_space=pltpu.SMEM),
        ],
        out_specs=pl.BlockSpec((GXBLK, G), lambda j: (j, 0)),
        out_shape=jax.ShapeDtypeStruct((G, G), jnp.float32),
    )(sfT, W)

    pr, idx = pl.pallas_call(
        _phi_body,
        grid=(B // BBLK,),
        in_specs=[
            pl.BlockSpec((20, BBLK), lambda j: (0, j)),
            pl.BlockSpec(memory_space=pltpu.SMEM),
        ],
        out_specs=[
            pl.BlockSpec((2, BBLK), lambda j: (0, j)),
            pl.BlockSpec((4, BBLK), lambda j: (0, j)),
        ],
        out_shape=[
            jax.ShapeDtypeStruct((2, B), jnp.float32),
            jax.ShapeDtypeStruct((4, B), jnp.int32),
        ],
    )(phiT, W)

    out = _make_sc_combine()(t_tab.reshape(P), idx, pr)
    return jnp.transpose(out, (1, 0))[:, :, None]


# speedup vs baseline: 1.9308x; 1.0843x over previous
import functools

import jax
import jax.numpy as jnp
from jax import lax
from jax.experimental import pallas as pl
from jax.experimental.pallas import tpu as pltpu
from jax.experimental.pallas import tpu_sc as plsc

S = 64
G = 128
P = G * G
F = 6
B = 16384
GXBLK = 32
BBLK = 2048
NW = 32
BPW = B // NW
L = 16


def _table_body(sf_ref, w_ref, t_ref):
    v = w_ref[0, 0] * sf_ref[:, 0]
    for f in range(1, F):
        v = v + w_ref[0, f] * sf_ref[:, f]
    m = jnp.max(v, axis=0)
    e = jnp.exp(v - m[None])
    z = jnp.sum(e, axis=0)
    num = jnp.sum(e * v, axis=0)
    t_ref[...] = num / z


def _phi_body(p_ref, w_ref, pr_ref, idx_ref):
    def row(k, c):
        return p_ref[k, c:c + 1]

    pr_l = w_ref[0, 0] * row(0, 0)
    pr_r = w_ref[0, 0] * row(0, 1)
    for f in range(1, F):
        pr_l = pr_l + w_ref[0, f] * row(f, 0)
        pr_r = pr_r + w_ref[0, f] * row(f, 1)
    pr_ref[...] = jnp.concatenate([pr_l, pr_r], axis=0)

    def flat_idx(c):
        return row(6, c).astype(jnp.int32) * G + row(7, c).astype(jnp.int32), \
               row(8, c).astype(jnp.int32) * G + row(9, c).astype(jnp.int32)

    ss_l, es_l = flat_idx(0)
    ss_r, es_r = flat_idx(1)
    idx_ref[...] = jnp.concatenate([ss_l, ss_r, es_l, es_r], axis=0)


def _sc_combine_body(t_hbm, idx_hbm, pr_hbm, out_hbm,
                     t_v, idx_v, pr_v, out_v, sem):
    wid = lax.axis_index("s") * 2 + lax.axis_index("c")
    base = wid * BPW
    c1 = pltpu.async_copy(t_hbm, t_v, sem)
    c2 = pltpu.async_copy(idx_hbm.at[:, pl.ds(base, BPW)], idx_v, sem)
    c3 = pltpu.async_copy(pr_hbm.at[:, pl.ds(base, BPW)], pr_v, sem)
    c1.wait()
    c2.wait()
    c3.wait()

    def body(c, carry):
        s = c * L
        v_ssl = plsc.load_gather(t_v, [idx_v[0, pl.ds(s, L)]])
        v_ssr = plsc.load_gather(t_v, [idx_v[1, pl.ds(s, L)]])
        v_esl = plsc.load_gather(t_v, [idx_v[2, pl.ds(s, L)]])
        v_esr = plsc.load_gather(t_v, [idx_v[3, pl.ds(s, L)]])
        d_l = pr_v[0, pl.ds(s, L)] + v_esl - v_ssl
        d_r = pr_v[1, pl.ds(s, L)] + v_esr - v_ssr
        z = d_l - d_r
        out_v[0, pl.ds(s, L)] = 1.0 / (1.0 + jnp.exp(-z))
        out_v[1, pl.ds(s, L)] = 1.0 / (1.0 + jnp.exp(z))
        return carry

    lax.fori_loop(0, BPW // L, body, 0)
    pltpu.sync_copy(out_v, out_hbm.at[:, pl.ds(base, BPW)])


@functools.cache
def _make_sc_combine():
    return functools.partial(
        pl.kernel,
        mesh=plsc.VectorSubcoreMesh(core_axis_name="c", subcore_axis_name="s"),
        out_type=jax.ShapeDtypeStruct((2, B), jnp.float32),
        compiler_params=pltpu.CompilerParams(needs_layout_passes=False),
        scratch_types=[
            pltpu.VMEM((P,), jnp.float32),
            pltpu.VMEM((4, BPW), jnp.int32),
            pltpu.VMEM((2, BPW), jnp.float32),
            pltpu.VMEM((2, BPW), jnp.float32),
            pltpu.SemaphoreType.DMA,
        ],
    )(_sc_combine_body)


def kernel(phi, succ_feats, W):
    sfT = jnp.transpose(succ_feats, (0, 3, 1, 2))
    phiT = jnp.transpose(phi, (2, 1, 0))

    t_tab = pl.pallas_call(
        _table_body,
        grid=(G // GXBLK,),
        in_specs=[
            pl.BlockSpec((S, F, GXBLK, G), lambda j: (0, 0, j, 0)),
            pl.BlockSpec(memory_space=pltpu.SMEM),
        ],
        out_specs=pl.BlockSpec((GXBLK, G), lambda j: (j, 0)),
        out_shape=jax.ShapeDtypeStruct((G, G), jnp.float32),
    )(sfT, W)

    pr, idx = pl.pallas_call(
        _phi_body,
        grid=(B // BBLK,),
        in_specs=[
            pl.BlockSpec((10, 2, BBLK), lambda j: (0, 0, j)),
            pl.BlockSpec(memory_space=pltpu.SMEM),
        ],
        out_specs=[
            pl.BlockSpec((2, BBLK), lambda j: (0, j)),
            pl.BlockSpec((4, BBLK), lambda j: (0, j)),
        ],
        out_shape=[
            jax.ShapeDtypeStruct((2, B), jnp.float32),
            jax.ShapeDtypeStruct((4, B), jnp.int32),
        ],
    )(phiT, W)

    out = _make_sc_combine()(t_tab.reshape(P), idx, pr)
    return jnp.transpose(out, (1, 0))[:, :, None]
